# baseline (speedup 1.0000x reference)
import functools
import math

import jax
import jax.numpy as jnp
from jax import lax
from jax.experimental import pallas as pl
from jax.experimental.pallas import tpu as pltpu

_TEMPERATURE = math.e


def _gating_kernel(k_active, x_ref, gw_ref, gb_ref, w_ref):
    x = x_ref[...]
    gw = gw_ref[...]
    gb = gb_ref[...]
    e = gw.shape[1]
    scores = (jnp.dot(x, gw, preferred_element_type=jnp.float32) + gb) / _TEMPERATURE
    scores = scores - jnp.max(scores, axis=-1, keepdims=True)
    ex = jnp.exp(scores)
    probs = ex / jnp.sum(ex, axis=-1, keepdims=True)
    ivec = lax.broadcasted_iota(jnp.int32, (1, e), 1)
    rank = jnp.zeros(probs.shape, jnp.int32)
    for j in range(e):
        pj = probs[:, j:j + 1]
        beats = jnp.logical_or(pj > probs,
                               jnp.logical_and(pj == probs, j < ivec))
        rank = rank + beats.astype(jnp.int32)
    mask = (rank < k_active).astype(jnp.float32)
    w = probs * mask
    w_ref[...] = w / (jnp.sum(w, axis=-1, keepdims=True) + 1e-8)


def _moe_kernel(bt, x_ref, w_ref, W1_ref, b1_ref, W2_ref, b2_ref, out_ref):
    e = pl.program_id(0)
    t = pl.program_id(1)
    w1 = W1_ref[0]
    w2 = W2_ref[0]
    b1 = b1_ref[0]
    n_e = w_ref.shape[1]
    onehot = (lax.broadcasted_iota(jnp.int32, (1, n_e), 1) == e).astype(jnp.float32)

    hb = bt // 2
    parts = []
    for i in range(2):
        sl = pl.ds(t * bt + i * hb, hb)
        x_blk = x_ref[pl.ds(i * hb, hb), :]
        h = jnp.dot(x_blk, w1, preferred_element_type=jnp.float32)
        h = jnp.maximum(h + b1, 0.0)
        o = jnp.dot(h, w2, preferred_element_type=jnp.float32)
        w_blk = w_ref[sl, :]
        w_col = jnp.sum(w_blk * onehot, axis=1, keepdims=True)
        parts.append((sl, w_blk, w_col * o))

    @pl.when(e == 0)
    def _init():
        for sl, w_blk, contrib in parts:
            out_ref[sl, :] = contrib + jnp.dot(
                w_blk, b2_ref[...], preferred_element_type=jnp.float32)

    @pl.when(e > 0)
    def _acc():
        for sl, _, contrib in parts:
            out_ref[sl, :] = out_ref[sl, :] + contrib


def kernel(x, gate_W, gate_b, W1, b1, W2, b2):
    T, D = x.shape
    E = gate_W.shape[1]
    H = W1.shape[2]
    O = W2.shape[2]
    k_active = max(1, int(E * 0.7))

    bt_gate = min(T, 1024)
    weights = pl.pallas_call(
        functools.partial(_gating_kernel, k_active),
        grid=(T // bt_gate,),
        in_specs=[
            pl.BlockSpec((bt_gate, D), lambda t: (t, 0)),
            pl.BlockSpec((D, E), lambda t: (0, 0)),
            pl.BlockSpec((1, E), lambda t: (0, 0)),
        ],
        out_specs=pl.BlockSpec((bt_gate, E), lambda t: (t, 0)),
        out_shape=jax.ShapeDtypeStruct((T, E), jnp.float32),
    )(x, gate_W, gate_b.reshape(1, E))

    bt = min(T, 512)
    n_bt = T // bt
    y = pl.pallas_call(
        functools.partial(_moe_kernel, bt),
        grid=(E, n_bt),
        in_specs=[
            pl.BlockSpec((bt, D), lambda e, t: (t, 0)),
            pl.BlockSpec((T, E), lambda e, t: (0, 0)),
            pl.BlockSpec((1, D, H), lambda e, t: (e, 0, 0)),
            pl.BlockSpec((1, 1, H), lambda e, t: (e, 0, 0)),
            pl.BlockSpec((1, H, O), lambda e, t: (e, 0, 0)),
            pl.BlockSpec((E, O), lambda e, t: (0, 0)),
        ],
        out_specs=pl.BlockSpec((T, O), lambda e, t: (0, 0)),
        out_shape=jax.ShapeDtypeStruct((T, O), jnp.float32),
        compiler_params=pltpu.CompilerParams(vmem_limit_bytes=112 * 1024 * 1024),
    )(x, weights, W1, b1.reshape(E, 1, H), W2, b2)
    return y


# speedup vs baseline: 1.0048x; 1.0048x over previous
import functools
import math

import jax
import jax.numpy as jnp
from jax import lax
from jax.experimental import pallas as pl
from jax.experimental.pallas import tpu as pltpu

_TEMPERATURE = math.e


def _gating_kernel(k_active, x_ref, gw_ref, gb_ref, w_ref):
    x = x_ref[...]
    gw = gw_ref[...]
    gb = gb_ref[...]
    e = gw.shape[1]
    scores = (jnp.dot(x, gw, preferred_element_type=jnp.float32) + gb) / _TEMPERATURE
    scores = scores - jnp.max(scores, axis=-1, keepdims=True)
    ex = jnp.exp(scores)
    probs = ex / jnp.sum(ex, axis=-1, keepdims=True)
    ivec = lax.broadcasted_iota(jnp.int32, (1, e), 1)
    rank = jnp.zeros(probs.shape, jnp.int32)
    for j in range(e):
        pj = probs[:, j:j + 1]
        beats = jnp.logical_or(pj > probs,
                               jnp.logical_and(pj == probs, j < ivec))
        rank = rank + beats.astype(jnp.int32)
    mask = (rank < k_active).astype(jnp.float32)
    w = probs * mask
    w_ref[...] = w / (jnp.sum(w, axis=-1, keepdims=True) + 1e-8)


def _moe_kernel(bt, x_ref, w_ref, W1_ref, b1_ref, W2_ref, b2_ref, out_ref):
    e = pl.program_id(0)
    t = pl.program_id(1)
    w1 = W1_ref[0]
    w2 = W2_ref[0]
    b1 = b1_ref[0]
    n_e = w_ref.shape[1]
    onehot = (lax.broadcasted_iota(jnp.int32, (1, n_e), 1) == e).astype(jnp.float32)

    hb = bt // 2
    parts = []
    for i in range(2):
        sl = pl.ds(t * bt + i * hb, hb)
        x_blk = x_ref[pl.ds(i * hb, hb), :]
        h = jnp.dot(x_blk, w1, preferred_element_type=jnp.float32)
        h = jnp.maximum(h + b1, 0.0)
        o = jnp.dot(h, w2, preferred_element_type=jnp.float32)
        w_blk = w_ref[sl, :]
        w_col = jnp.sum(w_blk * onehot, axis=1, keepdims=True)
        parts.append((sl, w_blk, w_col * o))

    @pl.when(e == 0)
    def _init():
        for sl, w_blk, contrib in parts:
            out_ref[sl, :] = contrib + jnp.dot(
                w_blk, b2_ref[...], preferred_element_type=jnp.float32)

    @pl.when(e > 0)
    def _acc():
        for sl, _, contrib in parts:
            out_ref[sl, :] = out_ref[sl, :] + contrib


def kernel(x, gate_W, gate_b, W1, b1, W2, b2):
    T, D = x.shape
    E = gate_W.shape[1]
    H = W1.shape[2]
    O = W2.shape[2]
    k_active = max(1, int(E * 0.7))

    bt_gate = min(T, 512)
    weights = pl.pallas_call(
        functools.partial(_gating_kernel, k_active),
        grid=(T // bt_gate,),
        in_specs=[
            pl.BlockSpec((bt_gate, D), lambda t: (t, 0)),
            pl.BlockSpec((D, E), lambda t: (0, 0)),
            pl.BlockSpec((1, E), lambda t: (0, 0)),
        ],
        out_specs=pl.BlockSpec((bt_gate, E), lambda t: (t, 0)),
        out_shape=jax.ShapeDtypeStruct((T, E), jnp.float32),
    )(x, gate_W, gate_b.reshape(1, E))

    bt = min(T, 512)
    n_bt = T // bt
    y = pl.pallas_call(
        functools.partial(_moe_kernel, bt),
        grid=(E, n_bt),
        in_specs=[
            pl.BlockSpec((bt, D), lambda e, t: (t, 0)),
            pl.BlockSpec((T, E), lambda e, t: (0, 0)),
            pl.BlockSpec((1, D, H), lambda e, t: (e, 0, 0)),
            pl.BlockSpec((1, 1, H), lambda e, t: (e, 0, 0)),
            pl.BlockSpec((1, H, O), lambda e, t: (e, 0, 0)),
            pl.BlockSpec((E, O), lambda e, t: (0, 0)),
        ],
        out_specs=pl.BlockSpec((T, O), lambda e, t: (0, 0)),
        out_shape=jax.ShapeDtypeStruct((T, O), jnp.float32),
        compiler_params=pltpu.CompilerParams(vmem_limit_bytes=112 * 1024 * 1024),
    )(x, weights, W1, b1.reshape(E, 1, H), W2, b2)
    return y
